# Initial kernel scaffold; baseline (speedup 1.0000x reference)
import jax
import jax.numpy as jnp
from jax.experimental import pallas as pl


def kernel(input_seq, label, mask, prototypes):
    raise NotImplementedError("write your pallas kernel here")


# speedup vs baseline: 3.9678x; 3.9678x over previous
import functools

import jax
import jax.numpy as jnp
from jax import lax
from jax.experimental import pallas as pl
from jax.experimental.pallas import tpu as pltpu
from jax.experimental.pallas import tpu_sc as plsc

B, P, S, D = 16, 512, 128, 64
K = S * D
PB = 128
NB = P // PB
R = 16
C = K // R


def _dist_body(x_ref, mb_ref, p_ref, dist_ref, idx_ref, minv, mina):
    i = pl.program_id(0)
    x = x_ref[...]
    mb = mb_ref[...]
    pb = p_ref[...]
    xm = x * mb
    cross = lax.dot_general(
        xm, pb, (((1,), (1,)), ((), ())),
        preferred_element_type=jnp.float32,
        precision=lax.Precision.HIGHEST)
    t2 = lax.dot_general(
        mb, pb * pb, (((1,), (1,)), ((), ())),
        preferred_element_type=jnp.float32,
        precision=lax.Precision.HIGHEST)
    x2m = jnp.sum(xm * x, axis=1, keepdims=True)
    dist = x2m + t2 - 2.0 * cross
    dist_ref[...] = dist

    lmin = jnp.min(dist, axis=1, keepdims=True)
    col = lax.broadcasted_iota(jnp.int32, (B, PB), 1)
    larg = jnp.min(jnp.where(dist == lmin, col, PB), axis=1,
                   keepdims=True) + i * PB

    @pl.when(i == 0)
    def _():
        minv[...] = lmin
        mina[...] = larg

    @pl.when(i > 0)
    def _():
        better = lmin < minv[...]
        mina[...] = jnp.where(better, larg, mina[...])
        minv[...] = jnp.where(better, lmin, minv[...])

    @pl.when(i == NB - 1)
    def _():
        idx_ref[...] = mina[...]


_dist_call = pl.pallas_call(
    _dist_body,
    grid=(NB,),
    in_specs=[
        pl.BlockSpec((B, K), lambda i: (0, 0)),
        pl.BlockSpec((B, K), lambda i: (0, 0)),
        pl.BlockSpec((PB, K), lambda i: (i, 0)),
    ],
    out_specs=[
        pl.BlockSpec((B, PB), lambda i: (0, i)),
        pl.BlockSpec((B, 1), lambda i: (0, 0)),
    ],
    out_shape=[
        jax.ShapeDtypeStruct((B, P), jnp.float32),
        jax.ShapeDtypeStruct((B, 1), jnp.int32),
    ],
    scratch_shapes=[
        pltpu.VMEM((B, 1), jnp.float32),
        pltpu.VMEM((B, 1), jnp.int32),
    ],
)


def _sc_gather_body(idx_hbm, tab_hbm, out_hbm, idxv, entv, rows, sem):
    cid = lax.axis_index("c")
    sid = lax.axis_index("s")
    wid = sid * 2 + cid
    r = wid // 2
    half = wid % 2
    pltpu.sync_copy(idx_hbm, idxv)
    entv[...] = idxv[...] * R + r
    pltpu.async_copy(
        tab_hbm.at[entv.at[pl.ds(half * 8, 8)]], rows, sem).wait()
    pltpu.sync_copy(rows, out_hbm.at[pl.ds(wid * 8, 8)])


@functools.lru_cache(maxsize=1)
def _sc_gather_call():
    mesh = plsc.VectorSubcoreMesh(
        core_axis_name="c", subcore_axis_name="s",
        num_cores=2, num_subcores=16)
    return pl.kernel(
        _sc_gather_body,
        out_type=jax.ShapeDtypeStruct((R * B, C), jnp.float32),
        mesh=mesh,
        scratch_types=[
            pltpu.VMEM((16,), jnp.int32),
            pltpu.VMEM((16,), jnp.int32),
            pltpu.VMEM((8, C), jnp.float32),
            pltpu.SemaphoreType.DMA,
        ],
    )


def kernel(input_seq, label, mask, prototypes):
    x2d = input_seq.reshape(B, K)
    mb = jnp.broadcast_to(mask[:, :, None], (B, S, D)).reshape(B, K)
    p2d = prototypes.reshape(P, K)
    dist, idx2 = _dist_call(x2d, mb, p2d)
    indices = idx2.reshape(B)
    out2 = _sc_gather_call()(indices, p2d.reshape(P * R, C))
    output_seq = out2.reshape(R, B, C).transpose(1, 0, 2).reshape(B, S, D)
    return (output_seq, input_seq, dist, indices, label, mask)


# speedup vs baseline: 5.8891x; 1.4842x over previous
import functools

import jax
import jax.numpy as jnp
from jax import lax
from jax.experimental import pallas as pl
from jax.experimental.pallas import tpu as pltpu
from jax.experimental.pallas import tpu_sc as plsc

B, P, S, D = 16, 512, 128, 64
K = S * D
PB = 128
NB = P // PB
R = 16
C = K // R


def _dist_body(x_ref, m_ref, p_ref, dist_ref, idx_ref, minv, mina):
    i = pl.program_id(0)
    x = x_ref[...]
    mb = pltpu.repeat(m_ref[...], D, axis=1)
    pb = p_ref[...]
    xm = x * mb
    cross = lax.dot_general(
        xm, pb, (((1,), (1,)), ((), ())),
        preferred_element_type=jnp.float32,
        precision=lax.Precision.HIGHEST)
    t2 = lax.dot_general(
        mb, pb * pb, (((1,), (1,)), ((), ())),
        preferred_element_type=jnp.float32,
        precision=lax.Precision.HIGHEST)
    x2m = jnp.sum(xm * x, axis=1, keepdims=True)
    dist = x2m + t2 - 2.0 * cross
    dist_ref[...] = dist

    lmin = jnp.min(dist, axis=1, keepdims=True)
    col = lax.broadcasted_iota(jnp.int32, (B, PB), 1)
    larg = jnp.min(jnp.where(dist == lmin, col, P), axis=1,
                   keepdims=True) + i * PB

    @pl.when(i == 0)
    def _():
        minv[...] = lmin
        mina[...] = larg

    @pl.when(i > 0)
    def _():
        better = lmin < minv[...]
        mina[...] = jnp.where(better, larg, mina[...])
        minv[...] = jnp.where(better, lmin, minv[...])

    @pl.when(i == NB - 1)
    def _():
        idx_ref[...] = mina[...]


_dist_call = pl.pallas_call(
    _dist_body,
    grid=(NB,),
    in_specs=[
        pl.BlockSpec((B, K), lambda i: (0, 0)),
        pl.BlockSpec((B, S), lambda i: (0, 0)),
        pl.BlockSpec((PB, K), lambda i: (i, 0)),
    ],
    out_specs=[
        pl.BlockSpec((B, PB), lambda i: (0, i)),
        pl.BlockSpec((B, 1), lambda i: (0, 0)),
    ],
    out_shape=[
        jax.ShapeDtypeStruct((B, P), jnp.float32),
        jax.ShapeDtypeStruct((B, 1), jnp.int32),
    ],
    scratch_shapes=[
        pltpu.VMEM((B, 1), jnp.float32),
        pltpu.VMEM((B, 1), jnp.int32),
    ],
)


def _sc_gather_body(idx_hbm, tab_hbm, out_hbm, idxv, entv, rows, sem):
    cid = lax.axis_index("c")
    sid = lax.axis_index("s")
    wid = sid * 2 + cid
    r = wid // 2
    half = wid % 2
    pltpu.sync_copy(idx_hbm, idxv)
    entv[...] = idxv[...] * R + r
    pltpu.async_copy(
        tab_hbm.at[entv.at[pl.ds(half * 8, 8)]], rows, sem).wait()
    pltpu.sync_copy(rows, out_hbm.at[pl.ds(wid * 8, 8)])


@functools.lru_cache(maxsize=1)
def _sc_gather_call():
    mesh = plsc.VectorSubcoreMesh(
        core_axis_name="c", subcore_axis_name="s",
        num_cores=2, num_subcores=16)
    return pl.kernel(
        _sc_gather_body,
        out_type=jax.ShapeDtypeStruct((R * B, C), jnp.float32),
        mesh=mesh,
        scratch_types=[
            pltpu.VMEM((16,), jnp.int32),
            pltpu.VMEM((16,), jnp.int32),
            pltpu.VMEM((8, C), jnp.float32),
            pltpu.SemaphoreType.DMA,
        ],
    )


def kernel(input_seq, label, mask, prototypes):
    xT = input_seq.transpose(0, 2, 1).reshape(B, K)
    pT = prototypes.transpose(0, 2, 1).reshape(P, K)
    dist, idx2 = _dist_call(xT, mask, pT)
    indices = idx2.reshape(B)
    out2 = _sc_gather_call()(indices, pT.reshape(P * R, C))
    output_seq = (out2.reshape(R, B, C).transpose(1, 0, 2)
                  .reshape(B, D, S).transpose(0, 2, 1))
    return (output_seq, input_seq, dist, indices, label, mask)


# speedup vs baseline: 6.7757x; 1.1506x over previous
import functools

import jax
import jax.numpy as jnp
from jax import lax
from jax.experimental import pallas as pl
from jax.experimental.pallas import tpu as pltpu
from jax.experimental.pallas import tpu_sc as plsc

B, P, S, D = 16, 512, 128, 64
K = S * D
PB = 128
NB = P // PB


def _dist_body(x_ref, m_ref, p_ref, dist_ref, idx_ref, minv, mina):
    i = pl.program_id(0)
    x = x_ref[...]
    mb = pltpu.repeat(m_ref[...], D, axis=1)
    pb = p_ref[...]
    xm = x * mb
    cross = lax.dot_general(
        xm, pb, (((1,), (1,)), ((), ())),
        preferred_element_type=jnp.float32,
        precision=lax.Precision.HIGHEST)
    t2 = lax.dot_general(
        mb, pb * pb, (((1,), (1,)), ((), ())),
        preferred_element_type=jnp.float32,
        precision=lax.Precision.HIGHEST)
    x2m = jnp.sum(xm * x, axis=1, keepdims=True)
    dist = x2m + t2 - 2.0 * cross
    dist_ref[...] = dist

    lmin = jnp.min(dist, axis=1, keepdims=True)
    col = lax.broadcasted_iota(jnp.int32, (B, PB), 1)
    larg = jnp.min(jnp.where(dist == lmin, col, P), axis=1,
                   keepdims=True) + i * PB

    @pl.when(i == 0)
    def _():
        minv[...] = lmin
        mina[...] = larg

    @pl.when(i > 0)
    def _():
        better = lmin < minv[...]
        mina[...] = jnp.where(better, larg, mina[...])
        minv[...] = jnp.where(better, lmin, minv[...])

    @pl.when(i == NB - 1)
    def _():
        idx_ref[...] = mina[...]


_dist_call = pl.pallas_call(
    _dist_body,
    grid=(NB,),
    in_specs=[
        pl.BlockSpec((B, K), lambda i: (0, 0)),
        pl.BlockSpec((B, S), lambda i: (0, 0)),
        pl.BlockSpec((PB, K), lambda i: (i, 0)),
    ],
    out_specs=[
        pl.BlockSpec((B, PB), lambda i: (0, i)),
        pl.BlockSpec((B, 1), lambda i: (0, 0)),
    ],
    out_shape=[
        jax.ShapeDtypeStruct((B, P), jnp.float32),
        jax.ShapeDtypeStruct((B, 1), jnp.int32),
    ],
    scratch_shapes=[
        pltpu.VMEM((B, 1), jnp.float32),
        pltpu.VMEM((B, 1), jnp.int32),
    ],
)


def _sc_gather_body(idx_hbm, tab_hbm, out_hbm, idxv, entv, rows, sem):
    cid = lax.axis_index("c")
    sid = lax.axis_index("s")
    wid = sid * 2 + cid
    pltpu.sync_copy(idx_hbm, idxv)
    base = idxv[...] * D
    entv[pl.ds(0, 16)] = base + 2 * wid
    entv[pl.ds(16, 16)] = base + 2 * wid + 1
    pltpu.async_copy(tab_hbm.at[entv], rows, sem).wait()
    pltpu.sync_copy(rows, out_hbm.at[pl.ds(wid * 32, 32)])


@functools.lru_cache(maxsize=1)
def _sc_gather_call():
    mesh = plsc.VectorSubcoreMesh(
        core_axis_name="c", subcore_axis_name="s",
        num_cores=2, num_subcores=16)
    return pl.kernel(
        _sc_gather_body,
        out_type=jax.ShapeDtypeStruct((D * B, S), jnp.float32),
        mesh=mesh,
        scratch_types=[
            pltpu.VMEM((16,), jnp.int32),
            pltpu.VMEM((32,), jnp.int32),
            pltpu.VMEM((32, S), jnp.float32),
            pltpu.SemaphoreType.DMA,
        ],
    )


def kernel(input_seq, label, mask, prototypes):
    xT = input_seq.transpose(0, 2, 1).reshape(B, K)
    pT = prototypes.transpose(0, 2, 1).reshape(P, K)
    dist, idx2 = _dist_call(xT, mask, pT)
    indices = idx2.reshape(B)
    out2 = _sc_gather_call()(indices, pT.reshape(P * D, S))
    output_seq = out2.reshape(D, B, S).transpose(1, 2, 0)
    return (output_seq, input_seq, dist, indices, label, mask)


# speedup vs baseline: 10.3994x; 1.5348x over previous
import functools

import jax
import jax.numpy as jnp
from jax import lax
from jax.experimental import pallas as pl
from jax.experimental.pallas import tpu as pltpu
from jax.experimental.pallas import tpu_sc as plsc

B, P, S, D = 16, 512, 128, 64
K = S * D
PB = 128
NB = P // PB


def _dist_body(x_ref, m_ref, p_ref, dist_ref, idx_ref, minv, mina):
    i = pl.program_id(0)
    x = x_ref[...]
    mb = pltpu.repeat(m_ref[...], D, axis=1)
    pb = p_ref[...].reshape(PB, K)
    xm = x * mb
    cross = lax.dot_general(
        xm, pb, (((1,), (1,)), ((), ())),
        preferred_element_type=jnp.float32,
        precision=lax.Precision.HIGHEST)
    q = pb[:, 0:S] * pb[:, 0:S]
    for d in range(1, D):
        sl = pb[:, d * S:(d + 1) * S]
        q = q + sl * sl
    t2 = lax.dot_general(
        m_ref[...], q, (((1,), (1,)), ((), ())),
        preferred_element_type=jnp.float32,
        precision=lax.Precision.HIGHEST)
    x2m = jnp.sum(xm * x, axis=1, keepdims=True)
    dist = x2m + t2 - 2.0 * cross
    dist_ref[...] = dist

    lmin = jnp.min(dist, axis=1, keepdims=True)
    col = lax.broadcasted_iota(jnp.int32, (B, PB), 1)
    larg = jnp.min(jnp.where(dist == lmin, col, P), axis=1,
                   keepdims=True) + i * PB

    @pl.when(i == 0)
    def _():
        minv[...] = lmin
        mina[...] = larg

    @pl.when(i > 0)
    def _():
        better = lmin < minv[...]
        mina[...] = jnp.where(better, larg, mina[...])
        minv[...] = jnp.where(better, lmin, minv[...])

    @pl.when(i == NB - 1)
    def _():
        idx_ref[...] = mina[...]


_dist_call = pl.pallas_call(
    _dist_body,
    grid=(NB,),
    in_specs=[
        pl.BlockSpec((B, K), lambda i: (0, 0)),
        pl.BlockSpec((B, S), lambda i: (0, 0)),
        pl.BlockSpec((PB, D, S), lambda i: (i, 0, 0)),
    ],
    out_specs=[
        pl.BlockSpec((B, PB), lambda i: (0, i)),
        pl.BlockSpec((B, 1), lambda i: (0, 0)),
    ],
    out_shape=[
        jax.ShapeDtypeStruct((B, P), jnp.float32),
        jax.ShapeDtypeStruct((B, 1), jnp.int32),
    ],
    scratch_shapes=[
        pltpu.VMEM((B, 1), jnp.float32),
        pltpu.VMEM((B, 1), jnp.int32),
    ],
)


def _sc_gather_body(idx_hbm, tab_hbm, out_hbm, idxv, entv, rows, sem):
    cid = lax.axis_index("c")
    sid = lax.axis_index("s")
    wid = sid * 2 + cid
    pltpu.sync_copy(idx_hbm, idxv)
    base = idxv[...] * D
    entv[pl.ds(0, 16)] = base + 2 * wid
    entv[pl.ds(16, 16)] = base + 2 * wid + 1
    pltpu.async_copy(tab_hbm.at[entv], rows, sem).wait()
    pltpu.sync_copy(rows, out_hbm.at[pl.ds(wid * 32, 32)])


@functools.lru_cache(maxsize=1)
def _sc_gather_call():
    mesh = plsc.VectorSubcoreMesh(
        core_axis_name="c", subcore_axis_name="s",
        num_cores=2, num_subcores=16)
    return pl.kernel(
        _sc_gather_body,
        out_type=jax.ShapeDtypeStruct((D * B, S), jnp.float32),
        mesh=mesh,
        scratch_types=[
            pltpu.VMEM((16,), jnp.int32),
            pltpu.VMEM((32,), jnp.int32),
            pltpu.VMEM((32, S), jnp.float32),
            pltpu.SemaphoreType.DMA,
        ],
    )


def kernel(input_seq, label, mask, prototypes):
    xT = input_seq.transpose(0, 2, 1).reshape(B, K)
    pT = prototypes.transpose(0, 2, 1).reshape(P, K)
    dist, idx2 = _dist_call(xT, mask, prototypes.transpose(0, 2, 1))
    indices = idx2.reshape(B)
    out2 = _sc_gather_call()(indices, pT.reshape(P * D, S))
    output_seq = out2.reshape(D, B, S).transpose(1, 2, 0)
    return (output_seq, input_seq, dist, indices, label, mask)


# speedup vs baseline: 10.5949x; 1.0188x over previous
import functools

import jax
import jax.numpy as jnp
from jax import lax
from jax.experimental import pallas as pl
from jax.experimental.pallas import tpu as pltpu
from jax.experimental.pallas import tpu_sc as plsc

B, P, S, D = 16, 512, 128, 64
K = S * D
PB = 128
NB = P // PB


def _dist_body(x_ref, m_ref, p_hbm, dist_ref, idx_ref, pbuf, sems, minv,
               mina):
    i = pl.program_id(0)
    slot = lax.rem(i, 2)
    nslot = lax.rem(i + 1, 2)

    @pl.when(i == 0)
    def _():
        pltpu.make_async_copy(
            p_hbm.at[pl.ds(0, PB)], pbuf.at[0], sems.at[0]).start()

    @pl.when(i + 1 < NB)
    def _():
        pltpu.make_async_copy(
            p_hbm.at[pl.ds((i + 1) * PB, PB)], pbuf.at[nslot],
            sems.at[nslot]).start()

    pltpu.make_async_copy(
        p_hbm.at[pl.ds(i * PB, PB)], pbuf.at[slot], sems.at[slot]).wait()

    x = x_ref[...].reshape(B, K)
    mb = pltpu.repeat(m_ref[...], D, axis=1)
    pb = pbuf[slot].reshape(PB, K)
    xm = x * mb
    cross = lax.dot_general(
        xm, pb, (((1,), (1,)), ((), ())),
        preferred_element_type=jnp.float32,
        precision=lax.Precision.HIGHEST)
    q = pb[:, 0:S] * pb[:, 0:S]
    for d in range(1, D):
        sl = pb[:, d * S:(d + 1) * S]
        q = q + sl * sl
    t2 = lax.dot_general(
        m_ref[...], q, (((1,), (1,)), ((), ())),
        preferred_element_type=jnp.float32,
        precision=lax.Precision.HIGHEST)
    x2m = jnp.sum(xm * x, axis=1, keepdims=True)
    dist = x2m + t2 - 2.0 * cross
    dist_ref[...] = dist

    lmin = jnp.min(dist, axis=1, keepdims=True)
    col = lax.broadcasted_iota(jnp.int32, (B, PB), 1)
    larg = jnp.min(jnp.where(dist == lmin, col, P), axis=1,
                   keepdims=True) + i * PB

    @pl.when(i == 0)
    def _():
        minv[...] = lmin
        mina[...] = larg

    @pl.when(i > 0)
    def _():
        better = lmin < minv[...]
        mina[...] = jnp.where(better, larg, mina[...])
        minv[...] = jnp.where(better, lmin, minv[...])

    @pl.when(i == NB - 1)
    def _():
        idx_ref[...] = mina[...]


_dist_call = pl.pallas_call(
    _dist_body,
    grid=(NB,),
    in_specs=[
        pl.BlockSpec((B, D, S), lambda i: (0, 0, 0)),
        pl.BlockSpec((B, S), lambda i: (0, 0)),
        pl.BlockSpec(memory_space=pltpu.MemorySpace.HBM),
    ],
    out_specs=[
        pl.BlockSpec((B, PB), lambda i: (0, i)),
        pl.BlockSpec((B, 1), lambda i: (0, 0)),
    ],
    out_shape=[
        jax.ShapeDtypeStruct((B, P), jnp.float32),
        jax.ShapeDtypeStruct((B, 1), jnp.int32),
    ],
    scratch_shapes=[
        pltpu.VMEM((2, PB, D, S), jnp.float32),
        pltpu.SemaphoreType.DMA((2,)),
        pltpu.VMEM((B, 1), jnp.float32),
        pltpu.VMEM((B, 1), jnp.int32),
    ],
)


def _sc_gather_body(idx_hbm, tab_hbm, out_hbm, idxv, entv, rows, sem):
    cid = lax.axis_index("c")
    sid = lax.axis_index("s")
    wid = sid * 2 + cid
    pltpu.sync_copy(idx_hbm, idxv)
    base = idxv[...] * D
    entv[pl.ds(0, 16)] = base + 2 * wid
    entv[pl.ds(16, 16)] = base + 2 * wid + 1
    pltpu.async_copy(tab_hbm.at[entv], rows, sem).wait()
    pltpu.sync_copy(rows, out_hbm.at[pl.ds(wid * 32, 32)])


@functools.lru_cache(maxsize=1)
def _sc_gather_call():
    mesh = plsc.VectorSubcoreMesh(
        core_axis_name="c", subcore_axis_name="s",
        num_cores=2, num_subcores=16)
    return pl.kernel(
        _sc_gather_body,
        out_type=jax.ShapeDtypeStruct((D * B, S), jnp.float32),
        mesh=mesh,
        scratch_types=[
            pltpu.VMEM((16,), jnp.int32),
            pltpu.VMEM((32,), jnp.int32),
            pltpu.VMEM((32, S), jnp.float32),
            pltpu.SemaphoreType.DMA,
        ],
    )


def kernel(input_seq, label, mask, prototypes):
    pT = prototypes.transpose(0, 2, 1)
    dist, idx2 = _dist_call(input_seq.transpose(0, 2, 1), mask, pT)
    indices = idx2.reshape(B)
    out2 = _sc_gather_call()(indices, pT.reshape(P * D, S))
    output_seq = out2.reshape(D, B, S).transpose(1, 2, 0)
    return (output_seq, input_seq, dist, indices, label, mask)


# speedup vs baseline: 10.6175x; 1.0021x over previous
import functools

import jax
import jax.numpy as jnp
from jax import lax
from jax.experimental import pallas as pl
from jax.experimental.pallas import tpu as pltpu
from jax.experimental.pallas import tpu_sc as plsc

B, P, S, D = 16, 512, 128, 64
K = S * D
PB = 128
NB = P // PB


def _dist_body(x_ref, m_ref, p_hbm, dist_ref, idx_ref, pbuf, sems, minv,
               mina):
    i = pl.program_id(0)
    slot = lax.rem(i, 2)
    nslot = lax.rem(i + 1, 2)

    @pl.when(i == 0)
    def _():
        pltpu.make_async_copy(
            p_hbm.at[pl.ds(0, PB)], pbuf.at[0], sems.at[0]).start()

    @pl.when(i + 1 < NB)
    def _():
        pltpu.make_async_copy(
            p_hbm.at[pl.ds((i + 1) * PB, PB)], pbuf.at[nslot],
            sems.at[nslot]).start()

    pltpu.make_async_copy(
        p_hbm.at[pl.ds(i * PB, PB)], pbuf.at[slot], sems.at[slot]).wait()

    x = x_ref[...].reshape(B, K)
    mb = pltpu.repeat(m_ref[...], D, axis=1)
    pb = pbuf[slot].reshape(PB, K)
    xm = x * mb
    cross = lax.dot_general(
        xm, pb, (((1,), (1,)), ((), ())),
        preferred_element_type=jnp.float32,
        precision=lax.Precision.HIGHEST)
    q = pb[:, 0:S] * pb[:, 0:S]
    for d in range(1, D):
        sl = pb[:, d * S:(d + 1) * S]
        q = q + sl * sl
    t2 = lax.dot_general(
        m_ref[...], q, (((1,), (1,)), ((), ())),
        preferred_element_type=jnp.float32,
        precision=lax.Precision.HIGHEST)
    x2m = jnp.sum(xm * x, axis=1, keepdims=True)
    dist = x2m + t2 - 2.0 * cross
    dist_ref[...] = dist

    lmin = jnp.min(dist, axis=1, keepdims=True)
    col = lax.broadcasted_iota(jnp.int32, (B, PB), 1)
    larg = jnp.min(jnp.where(dist == lmin, col, P), axis=1,
                   keepdims=True) + i * PB

    @pl.when(i == 0)
    def _():
        minv[...] = lmin
        mina[...] = larg

    @pl.when(i > 0)
    def _():
        better = lmin < minv[...]
        mina[...] = jnp.where(better, larg, mina[...])
        minv[...] = jnp.where(better, lmin, minv[...])

    @pl.when(i == NB - 1)
    def _():
        idx_ref[...] = mina[...]


_dist_call = pl.pallas_call(
    _dist_body,
    grid=(NB,),
    in_specs=[
        pl.BlockSpec((B, D, S), lambda i: (0, 0, 0)),
        pl.BlockSpec((B, S), lambda i: (0, 0)),
        pl.BlockSpec(memory_space=pltpu.MemorySpace.HBM),
    ],
    out_specs=[
        pl.BlockSpec((B, PB), lambda i: (0, i)),
        pl.BlockSpec((B, 1), lambda i: (0, 0)),
    ],
    out_shape=[
        jax.ShapeDtypeStruct((B, P), jnp.float32),
        jax.ShapeDtypeStruct((B, 1), jnp.int32),
    ],
    scratch_shapes=[
        pltpu.VMEM((2, PB, D, S), jnp.float32),
        pltpu.SemaphoreType.DMA((2,)),
        pltpu.VMEM((B, 1), jnp.float32),
        pltpu.VMEM((B, 1), jnp.int32),
    ],
)


def _sc_gather_body(idx_hbm, tab_hbm, out_hbm, idxv, entv, rows, sem):
    cid = lax.axis_index("c")
    sid = lax.axis_index("s")
    wid = sid * 2 + cid
    pltpu.sync_copy(idx_hbm, idxv)
    base = idxv[...] * D
    entv[pl.ds(0, 16)] = base + 2 * wid
    entv[pl.ds(16, 16)] = base + 2 * wid + 1
    pltpu.async_copy(tab_hbm.at[entv], rows, sem).wait()
    pltpu.sync_copy(rows, out_hbm.at[pl.ds(wid * 32, 32)])


@functools.lru_cache(maxsize=1)
def _sc_gather_call():
    mesh = plsc.VectorSubcoreMesh(
        core_axis_name="c", subcore_axis_name="s",
        num_cores=2, num_subcores=16)
    return pl.kernel(
        _sc_gather_body,
        out_type=jax.ShapeDtypeStruct((D * B, S), jnp.float32),
        mesh=mesh,
        scratch_types=[
            pltpu.VMEM((16,), jnp.int32),
            pltpu.VMEM((32,), jnp.int32),
            pltpu.VMEM((32, S), jnp.float32),
            pltpu.SemaphoreType.DMA,
        ],
    )


def kernel(input_seq, label, mask, prototypes):
    pT = prototypes.transpose(0, 2, 1)
    p_in = pltpu.with_memory_space_constraint(pT, pltpu.MemorySpace.HBM)
    dist, idx2 = _dist_call(input_seq.transpose(0, 2, 1), mask, p_in)
    indices = idx2.reshape(B)
    out2 = _sc_gather_call()(indices, pT.reshape(P * D, S))
    output_seq = out2.reshape(D, B, S).transpose(1, 2, 0)
    return (output_seq, input_seq, dist, indices, label, mask)


# speedup vs baseline: 11.7714x; 1.1087x over previous
import functools

import jax
import jax.numpy as jnp
from jax import lax
from jax.experimental import pallas as pl
from jax.experimental.pallas import tpu as pltpu
from jax.experimental.pallas import tpu_sc as plsc

B, P, S, D = 16, 512, 128, 64
K = S * D
PB = 128
NB = P // PB


def _dist_body(x_ref, m_ref, p_hbm, dist_ref, idx_ref, pbuf, sems, minv,
               mina):
    i = pl.program_id(0)
    slot = lax.rem(i, 2)
    nslot = lax.rem(i + 1, 2)

    @pl.when(i == 0)
    def _():
        pltpu.make_async_copy(
            p_hbm.at[pl.ds(0, PB)], pbuf.at[0], sems.at[0]).start()

    @pl.when(i + 1 < NB)
    def _():
        pltpu.make_async_copy(
            p_hbm.at[pl.ds((i + 1) * PB, PB)], pbuf.at[nslot],
            sems.at[nslot]).start()

    pltpu.make_async_copy(
        p_hbm.at[pl.ds(i * PB, PB)], pbuf.at[slot], sems.at[slot]).wait()

    x = x_ref[...].reshape(B, K)
    mb = pltpu.repeat(m_ref[...], D, axis=1)
    pb = pbuf[slot].reshape(PB, K)
    xm = x * mb
    ph = pb.astype(jnp.bfloat16)
    plo = (pb - ph.astype(jnp.float32)).astype(jnp.bfloat16)
    xh = xm.astype(jnp.bfloat16)
    xlo = (xm - xh.astype(jnp.float32)).astype(jnp.bfloat16)

    def _dot(a, b):
        return lax.dot_general(a, b, (((1,), (1,)), ((), ())),
                               preferred_element_type=jnp.float32)

    cross = _dot(xh, ph) + (_dot(xh, plo) + _dot(xlo, ph))
    q = pb[:, 0:S] * pb[:, 0:S]
    for d in range(1, D):
        sl = pb[:, d * S:(d + 1) * S]
        q = q + sl * sl
    t2 = lax.dot_general(
        m_ref[...], q, (((1,), (1,)), ((), ())),
        preferred_element_type=jnp.float32,
        precision=lax.Precision.HIGHEST)
    x2m = jnp.sum(xm * x, axis=1, keepdims=True)
    dist = x2m + t2 - 2.0 * cross
    dist_ref[...] = dist

    lmin = jnp.min(dist, axis=1, keepdims=True)
    col = lax.broadcasted_iota(jnp.int32, (B, PB), 1)
    larg = jnp.min(jnp.where(dist == lmin, col, P), axis=1,
                   keepdims=True) + i * PB

    @pl.when(i == 0)
    def _():
        minv[...] = lmin
        mina[...] = larg

    @pl.when(i > 0)
    def _():
        better = lmin < minv[...]
        mina[...] = jnp.where(better, larg, mina[...])
        minv[...] = jnp.where(better, lmin, minv[...])

    @pl.when(i == NB - 1)
    def _():
        idx_ref[...] = mina[...]


_dist_call = pl.pallas_call(
    _dist_body,
    grid=(NB,),
    in_specs=[
        pl.BlockSpec((B, D, S), lambda i: (0, 0, 0)),
        pl.BlockSpec((B, S), lambda i: (0, 0)),
        pl.BlockSpec(memory_space=pltpu.MemorySpace.HBM),
    ],
    out_specs=[
        pl.BlockSpec((B, PB), lambda i: (0, i)),
        pl.BlockSpec((B, 1), lambda i: (0, 0)),
    ],
    out_shape=[
        jax.ShapeDtypeStruct((B, P), jnp.float32),
        jax.ShapeDtypeStruct((B, 1), jnp.int32),
    ],
    scratch_shapes=[
        pltpu.VMEM((2, PB, D, S), jnp.float32),
        pltpu.SemaphoreType.DMA((2,)),
        pltpu.VMEM((B, 1), jnp.float32),
        pltpu.VMEM((B, 1), jnp.int32),
    ],
)


def _sc_gather_body(idx_hbm, tab_hbm, out_hbm, idxv, entv, rows, sem):
    cid = lax.axis_index("c")
    sid = lax.axis_index("s")
    wid = sid * 2 + cid
    pltpu.sync_copy(idx_hbm, idxv)
    base = idxv[...] * D
    entv[pl.ds(0, 16)] = base + 2 * wid
    entv[pl.ds(16, 16)] = base + 2 * wid + 1
    pltpu.async_copy(tab_hbm.at[entv], rows, sem).wait()
    pltpu.sync_copy(rows, out_hbm.at[pl.ds(wid * 32, 32)])


@functools.lru_cache(maxsize=1)
def _sc_gather_call():
    mesh = plsc.VectorSubcoreMesh(
        core_axis_name="c", subcore_axis_name="s",
        num_cores=2, num_subcores=16)
    return pl.kernel(
        _sc_gather_body,
        out_type=jax.ShapeDtypeStruct((D * B, S), jnp.float32),
        mesh=mesh,
        compiler_params=pltpu.CompilerParams(skip_device_barrier=True),
        scratch_types=[
            pltpu.VMEM((16,), jnp.int32),
            pltpu.VMEM((32,), jnp.int32),
            pltpu.VMEM((32, S), jnp.float32),
            pltpu.SemaphoreType.DMA,
        ],
    )


def kernel(input_seq, label, mask, prototypes):
    pT = prototypes.transpose(0, 2, 1)
    p_in = pltpu.with_memory_space_constraint(pT, pltpu.MemorySpace.HBM)
    dist, idx2 = _dist_call(input_seq.transpose(0, 2, 1), mask, p_in)
    indices = idx2.reshape(B)
    out2 = _sc_gather_call()(indices, pT.reshape(P * D, S))
    output_seq = out2.reshape(D, B, S).transpose(1, 2, 0)
    return (output_seq, input_seq, dist, indices, label, mask)


# speedup vs baseline: 11.8009x; 1.0025x over previous
import functools

import jax
import jax.numpy as jnp
from jax import lax
from jax.experimental import pallas as pl
from jax.experimental.pallas import tpu as pltpu
from jax.experimental.pallas import tpu_sc as plsc

B, P, S, D = 16, 512, 128, 64
K = S * D
PB = 128
NB = P // PB


def _dist_body(x_ref, m_ref, p_hbm, dist_ref, idx_ref, pbuf, sems, minv,
               mina):
    i = pl.program_id(0)
    slot = lax.rem(i, 2)
    nslot = lax.rem(i + 1, 2)

    H = PB // 2

    def _start(blk, s):
        pltpu.make_async_copy(
            p_hbm.at[pl.ds(blk * PB, H)], pbuf.at[s, pl.ds(0, H)],
            sems.at[s, 0]).start()
        pltpu.make_async_copy(
            p_hbm.at[pl.ds(blk * PB + H, H)], pbuf.at[s, pl.ds(H, H)],
            sems.at[s, 1]).start()

    def _wait(blk, s):
        pltpu.make_async_copy(
            p_hbm.at[pl.ds(blk * PB, H)], pbuf.at[s, pl.ds(0, H)],
            sems.at[s, 0]).wait()
        pltpu.make_async_copy(
            p_hbm.at[pl.ds(blk * PB + H, H)], pbuf.at[s, pl.ds(H, H)],
            sems.at[s, 1]).wait()

    @pl.when(i == 0)
    def _():
        _start(0, 0)

    @pl.when(i + 1 < NB)
    def _():
        _start(i + 1, nslot)

    _wait(i, slot)

    x = x_ref[...].reshape(B, K)
    mb = pltpu.repeat(m_ref[...], D, axis=1)
    pb = pbuf[slot].reshape(PB, K)
    xm = x * mb
    ph = pb.astype(jnp.bfloat16)
    plo = (pb - ph.astype(jnp.float32)).astype(jnp.bfloat16)
    xh = xm.astype(jnp.bfloat16)
    xlo = (xm - xh.astype(jnp.float32)).astype(jnp.bfloat16)

    def _dot(a, b):
        return lax.dot_general(a, b, (((1,), (1,)), ((), ())),
                               preferred_element_type=jnp.float32)

    cross = _dot(xh, ph) + (_dot(xh, plo) + _dot(xlo, ph))
    q = pb[:, 0:S] * pb[:, 0:S]
    for d in range(1, D):
        sl = pb[:, d * S:(d + 1) * S]
        q = q + sl * sl
    t2 = lax.dot_general(
        m_ref[...], q, (((1,), (1,)), ((), ())),
        preferred_element_type=jnp.float32,
        precision=lax.Precision.HIGHEST)
    x2m = jnp.sum(xm * x, axis=1, keepdims=True)
    dist = x2m + t2 - 2.0 * cross
    dist_ref[...] = dist

    lmin = jnp.min(dist, axis=1, keepdims=True)
    col = lax.broadcasted_iota(jnp.int32, (B, PB), 1)
    larg = jnp.min(jnp.where(dist == lmin, col, P), axis=1,
                   keepdims=True) + i * PB

    @pl.when(i == 0)
    def _():
        minv[...] = lmin
        mina[...] = larg

    @pl.when(i > 0)
    def _():
        better = lmin < minv[...]
        mina[...] = jnp.where(better, larg, mina[...])
        minv[...] = jnp.where(better, lmin, minv[...])

    @pl.when(i == NB - 1)
    def _():
        idx_ref[...] = mina[...]


_dist_call = pl.pallas_call(
    _dist_body,
    grid=(NB,),
    in_specs=[
        pl.BlockSpec((B, D, S), lambda i: (0, 0, 0)),
        pl.BlockSpec((B, S), lambda i: (0, 0)),
        pl.BlockSpec(memory_space=pltpu.MemorySpace.HBM),
    ],
    out_specs=[
        pl.BlockSpec((B, PB), lambda i: (0, i)),
        pl.BlockSpec((B, 1), lambda i: (0, 0)),
    ],
    out_shape=[
        jax.ShapeDtypeStruct((B, P), jnp.float32),
        jax.ShapeDtypeStruct((B, 1), jnp.int32),
    ],
    scratch_shapes=[
        pltpu.VMEM((2, PB, D, S), jnp.float32),
        pltpu.SemaphoreType.DMA((2, 2)),
        pltpu.VMEM((B, 1), jnp.float32),
        pltpu.VMEM((B, 1), jnp.int32),
    ],
)


def _sc_gather_body(idx_hbm, tab_hbm, out_hbm, idxv, entv, rows, sem):
    cid = lax.axis_index("c")
    sid = lax.axis_index("s")
    wid = sid * 2 + cid
    pltpu.sync_copy(idx_hbm, idxv)
    base = idxv[...] * D
    entv[pl.ds(0, 16)] = base + 2 * wid
    entv[pl.ds(16, 16)] = base + 2 * wid + 1
    pltpu.async_copy(tab_hbm.at[entv], rows, sem).wait()
    pltpu.sync_copy(rows, out_hbm.at[pl.ds(wid * 32, 32)])


@functools.lru_cache(maxsize=1)
def _sc_gather_call():
    mesh = plsc.VectorSubcoreMesh(
        core_axis_name="c", subcore_axis_name="s",
        num_cores=2, num_subcores=16)
    return pl.kernel(
        _sc_gather_body,
        out_type=jax.ShapeDtypeStruct((D * B, S), jnp.float32),
        mesh=mesh,
        compiler_params=pltpu.CompilerParams(skip_device_barrier=True),
        scratch_types=[
            pltpu.VMEM((16,), jnp.int32),
            pltpu.VMEM((32,), jnp.int32),
            pltpu.VMEM((32, S), jnp.float32),
            pltpu.SemaphoreType.DMA,
        ],
    )


def kernel(input_seq, label, mask, prototypes):
    pT = prototypes.transpose(0, 2, 1)
    p_in = pltpu.with_memory_space_constraint(pT, pltpu.MemorySpace.HBM)
    dist, idx2 = _dist_call(input_seq.transpose(0, 2, 1), mask, p_in)
    indices = idx2.reshape(B)
    out2 = _sc_gather_call()(indices, pT.reshape(P * D, S))
    output_seq = out2.reshape(D, B, S).transpose(1, 2, 0)
    return (output_seq, input_seq, dist, indices, label, mask)


# speedup vs baseline: 12.3328x; 1.0451x over previous
import functools

import jax
import jax.numpy as jnp
from jax import lax
from jax.experimental import pallas as pl
from jax.experimental.pallas import tpu as pltpu
from jax.experimental.pallas import tpu_sc as plsc

B, P, S, D = 16, 512, 128, 64
K = S * D
PB = 128
NB = P // PB


def _dist_body(x_ref, m_ref, p_hbm, dist_ref, idx_ref, pbuf, sems, minv,
               mina):
    i = pl.program_id(0)
    slot = lax.rem(i, 2)
    nslot = lax.rem(i + 1, 2)

    H = PB // 2

    def _start(blk, s):
        pltpu.make_async_copy(
            p_hbm.at[pl.ds(blk * PB, H)], pbuf.at[s, pl.ds(0, H)],
            sems.at[s, 0]).start()
        pltpu.make_async_copy(
            p_hbm.at[pl.ds(blk * PB + H, H)], pbuf.at[s, pl.ds(H, H)],
            sems.at[s, 1]).start()

    def _wait(blk, s):
        pltpu.make_async_copy(
            p_hbm.at[pl.ds(blk * PB, H)], pbuf.at[s, pl.ds(0, H)],
            sems.at[s, 0]).wait()
        pltpu.make_async_copy(
            p_hbm.at[pl.ds(blk * PB + H, H)], pbuf.at[s, pl.ds(H, H)],
            sems.at[s, 1]).wait()

    @pl.when(i == 0)
    def _():
        _start(0, 0)

    @pl.when(i + 1 < NB)
    def _():
        _start(i + 1, nslot)

    _wait(i, slot)

    x = x_ref[...].reshape(B, K)
    mb = pltpu.repeat(m_ref[...], D, axis=1)
    pb = pbuf[slot].reshape(PB, K)
    xm = x * mb
    ph = pb.astype(jnp.bfloat16)
    plo = (pb - ph.astype(jnp.float32)).astype(jnp.bfloat16)
    xh = xm.astype(jnp.bfloat16)
    xlo = (xm - xh.astype(jnp.float32)).astype(jnp.bfloat16)

    def _dot(a, b):
        return lax.dot_general(a, b, (((1,), (1,)), ((), ())),
                               preferred_element_type=jnp.float32)

    cross = _dot(xh, ph) + (_dot(xh, plo) + _dot(xlo, ph))
    q = pb[:, 0:S] * pb[:, 0:S]
    for d in range(1, D):
        sl = pb[:, d * S:(d + 1) * S]
        q = q + sl * sl
    t2 = lax.dot_general(
        m_ref[...], q, (((1,), (1,)), ((), ())),
        preferred_element_type=jnp.float32,
        precision=lax.Precision.HIGHEST)
    x2m = jnp.sum(xm * x, axis=1, keepdims=True)
    dist = x2m + t2 - 2.0 * cross
    dist_ref[...] = dist

    lmin = jnp.min(dist, axis=1, keepdims=True)
    col = lax.broadcasted_iota(jnp.int32, (B, PB), 1)
    larg = jnp.min(jnp.where(dist == lmin, col, P), axis=1,
                   keepdims=True) + i * PB

    @pl.when(i == 0)
    def _():
        minv[...] = lmin
        mina[...] = larg

    @pl.when(i > 0)
    def _():
        better = lmin < minv[...]
        mina[...] = jnp.where(better, larg, mina[...])
        minv[...] = jnp.where(better, lmin, minv[...])

    @pl.when(i == NB - 1)
    def _():
        idx_ref[...] = mina[...]


_dist_call = pl.pallas_call(
    _dist_body,
    grid=(NB,),
    in_specs=[
        pl.BlockSpec((B, D, S), lambda i: (0, 0, 0)),
        pl.BlockSpec((B, S), lambda i: (0, 0)),
        pl.BlockSpec(memory_space=pltpu.MemorySpace.HBM),
    ],
    out_specs=[
        pl.BlockSpec((B, PB), lambda i: (0, i)),
        pl.BlockSpec((B, 1), lambda i: (0, 0)),
    ],
    out_shape=[
        jax.ShapeDtypeStruct((B, P), jnp.float32),
        jax.ShapeDtypeStruct((B, 1), jnp.int32),
    ],
    scratch_shapes=[
        pltpu.VMEM((2, PB, D, S), jnp.float32),
        pltpu.SemaphoreType.DMA((2, 2)),
        pltpu.VMEM((B, 1), jnp.float32),
        pltpu.VMEM((B, 1), jnp.int32),
    ],
)


def _sc_gather_body(idx_hbm, tab_hbm, out_hbm, idxv, entv, rows, sem):
    wid = lax.axis_index("s")
    pltpu.sync_copy(idx_hbm, idxv)
    base = idxv[...] * D
    entv[pl.ds(0, 16)] = base + 4 * wid
    entv[pl.ds(16, 16)] = base + 4 * wid + 1
    entv[pl.ds(32, 16)] = base + 4 * wid + 2
    entv[pl.ds(48, 16)] = base + 4 * wid + 3
    pltpu.async_copy(tab_hbm.at[entv], rows, sem).wait()
    pltpu.sync_copy(rows, out_hbm.at[pl.ds(wid * 64, 64)])


@functools.lru_cache(maxsize=1)
def _sc_gather_call():
    mesh = plsc.VectorSubcoreMesh(
        core_axis_name="c", subcore_axis_name="s",
        num_cores=1, num_subcores=16)
    return pl.kernel(
        _sc_gather_body,
        out_type=jax.ShapeDtypeStruct((D * B, S), jnp.float32),
        mesh=mesh,
        compiler_params=pltpu.CompilerParams(skip_device_barrier=True),
        scratch_types=[
            pltpu.VMEM((16,), jnp.int32),
            pltpu.VMEM((64,), jnp.int32),
            pltpu.VMEM((64, S), jnp.float32),
            pltpu.SemaphoreType.DMA,
        ],
    )


def kernel(input_seq, label, mask, prototypes):
    pT = prototypes.transpose(0, 2, 1)
    p_in = pltpu.with_memory_space_constraint(pT, pltpu.MemorySpace.HBM)
    dist, idx2 = _dist_call(input_seq.transpose(0, 2, 1), mask, p_in)
    indices = idx2.reshape(B)
    out2 = _sc_gather_call()(indices, pT.reshape(P * D, S))
    output_seq = out2.reshape(D, B, S).transpose(1, 2, 0)
    return (output_seq, input_seq, dist, indices, label, mask)
